# baseline (device time: 870403 ns/iter reference)
import jax
import jax.numpy as jnp
from jax import lax
from jax.experimental import pallas as pl
from jax.experimental.pallas import tpu as pltpu

M_HALF = 4096
D = 4096
CHUNK = 256
N_CHUNKS = M_HALF // CHUNK
EPS = 1e-6


def kernel(partial, gamma):
    gamma2d = gamma.reshape(1, D)

    def body(partial_ref, gamma_ref, out_ref,
             vmem_a, vmem_b, vmem_o,
             send_sem, recv_sem, sem_a, sem_b, sem_o):
        my_x = lax.axis_index("x")
        my_y = lax.axis_index("y")
        my_z = lax.axis_index("z")
        partner = (1 - my_x, my_y, my_z)

        barrier_sem = pltpu.get_barrier_semaphore()
        pl.semaphore_signal(barrier_sem, inc=1, device_id=partner,
                            device_id_type=pl.DeviceIdType.MESH)
        pl.semaphore_wait(barrier_sem, 1)

        rdma = pltpu.make_async_remote_copy(
            src_ref=partial_ref.at[0, pl.ds((1 - my_x) * M_HALF, M_HALF), :],
            dst_ref=out_ref,
            send_sem=send_sem,
            recv_sem=recv_sem,
            device_id=partner,
            device_id_type=pl.DeviceIdType.MESH,
        )
        rdma.start()
        rdma.wait()

        def chunk_body(c, carry):
            row0 = c * CHUNK
            cp_a = pltpu.make_async_copy(
                partial_ref.at[0, pl.ds(my_x * M_HALF + row0, CHUNK), :],
                vmem_a, sem_a)
            cp_b = pltpu.make_async_copy(
                out_ref.at[pl.ds(row0, CHUNK), :], vmem_b, sem_b)
            cp_a.start()
            cp_b.start()
            cp_a.wait()
            cp_b.wait()
            s = vmem_a[:, :] + vmem_b[:, :]
            r = lax.rsqrt(jnp.mean(s * s, axis=-1, keepdims=True) + EPS)
            vmem_o[:, :] = s * r * gamma_ref[:, :]
            cp_o = pltpu.make_async_copy(
                vmem_o, out_ref.at[pl.ds(row0, CHUNK), :], sem_o)
            cp_o.start()
            cp_o.wait()
            return carry

        lax.fori_loop(0, N_CHUNKS, chunk_body, 0)

    return pl.pallas_call(
        body,
        out_shape=jax.ShapeDtypeStruct((M_HALF, D), jnp.float32),
        in_specs=[
            pl.BlockSpec(memory_space=pl.ANY),
            pl.BlockSpec(memory_space=pltpu.MemorySpace.VMEM),
        ],
        out_specs=pl.BlockSpec(memory_space=pl.ANY),
        scratch_shapes=[
            pltpu.VMEM((CHUNK, D), jnp.float32),
            pltpu.VMEM((CHUNK, D), jnp.float32),
            pltpu.VMEM((CHUNK, D), jnp.float32),
            pltpu.SemaphoreType.DMA,
            pltpu.SemaphoreType.DMA,
            pltpu.SemaphoreType.DMA,
            pltpu.SemaphoreType.DMA,
            pltpu.SemaphoreType.DMA,
        ],
        compiler_params=pltpu.CompilerParams(collective_id=0),
    )(partial, gamma2d)


# device time: 479101 ns/iter; 1.8167x vs baseline; 1.8167x over previous
import numpy as np
import jax
import jax.numpy as jnp
from jax import lax
from jax.experimental import pallas as pl
from jax.experimental.pallas import tpu as pltpu

M_HALF = 4096
D = 4096
N_RING = 16
SL = M_HALF // N_RING
H_R = 8
H_L = 7
EPS = 1e-6

_RING = [
    (0, 0), (0, 1), (0, 2), (0, 3),
    (1, 3), (1, 2), (1, 1),
    (2, 1), (2, 2), (2, 3),
    (3, 3), (3, 2), (3, 1), (3, 0),
    (2, 0), (1, 0),
]
_POS = np.zeros(16, dtype=np.int32)
for _i, (_y, _z) in enumerate(_RING):
    _POS[_y * 4 + _z] = _i
_RING_Y = np.array([y for y, _ in _RING], dtype=np.int32)
_RING_Z = np.array([z for _, z in _RING], dtype=np.int32)


def kernel(partial, gamma):
    gamma2d = gamma.reshape(1, D)

    my_y = lax.axis_index("y")
    my_z = lax.axis_index("z")
    p = jnp.take(jnp.asarray(_POS), my_y * 4 + my_z)
    rp = (p + 1) % N_RING
    lp = (p + N_RING - 1) % N_RING
    scal = jnp.stack([
        p,
        jnp.take(jnp.asarray(_RING_Y), rp), jnp.take(jnp.asarray(_RING_Z), rp),
        jnp.take(jnp.asarray(_RING_Y), lp), jnp.take(jnp.asarray(_RING_Z), lp),
    ]).astype(jnp.int32)

    def body(partial_ref, gamma_ref, scal_ref, out_ref,
             vmem_my, vmem_recv, vmem_out,
             pa_send, pa_recv, sem_in, sem_out,
             r_send, r_recv, l_send, l_recv):
        my_x = lax.axis_index("x")
        my_y = lax.axis_index("y")
        my_z = lax.axis_index("z")
        p = scal_ref[0]
        right = (my_x, scal_ref[1], scal_ref[2])
        left = (my_x, scal_ref[3], scal_ref[4])
        partner = (1 - my_x, my_y, my_z)

        barrier_sem = pltpu.get_barrier_semaphore()
        for nbr in (partner, right, left):
            pl.semaphore_signal(barrier_sem, inc=1, device_id=nbr,
                                device_id_type=pl.DeviceIdType.MESH)
        pl.semaphore_wait(barrier_sem, 3)

        rdma_a = pltpu.make_async_remote_copy(
            src_ref=partial_ref.at[0, pl.ds((1 - my_x) * M_HALF + p * SL, SL), :],
            dst_ref=vmem_recv,
            send_sem=pa_send,
            recv_sem=pa_recv,
            device_id=partner,
            device_id_type=pl.DeviceIdType.MESH,
        )
        rdma_a.start()
        cp_in = pltpu.make_async_copy(
            partial_ref.at[0, pl.ds(my_x * M_HALF + p * SL, SL), :],
            vmem_my, sem_in)
        cp_in.start()
        cp_in.wait()
        rdma_a.wait()

        s = vmem_my[:, :] + vmem_recv[:, :]
        r = lax.rsqrt(jnp.mean(s * s, axis=-1, keepdims=True) + EPS)
        vmem_out[:, :] = s * r * gamma_ref[:, :]
        cp_out = pltpu.make_async_copy(
            vmem_out, out_ref.at[pl.ds(p * SL, SL), :], sem_out)
        cp_out.start()
        cp_out.wait()

        def ring_send(origin, sem_s, sem_r, dev):
            d = pltpu.make_async_remote_copy(
                src_ref=out_ref.at[pl.ds(origin * SL, SL), :],
                dst_ref=out_ref.at[pl.ds(origin * SL, SL), :],
                send_sem=sem_s, recv_sem=sem_r,
                device_id=dev, device_id_type=pl.DeviceIdType.MESH,
            )
            d.start()
            return d

        def ring_recv_wait(origin, sem_r):
            d = pltpu.make_async_remote_copy(
                src_ref=out_ref.at[pl.ds(origin * SL, SL), :],
                dst_ref=out_ref.at[pl.ds(origin * SL, SL), :],
                send_sem=pa_send, recv_sem=sem_r,
                device_id=right, device_id_type=pl.DeviceIdType.MESH,
            )
            d.wait_recv()

        send_descs = []
        for h in range(H_R):
            if h > 0:
                ring_recv_wait((p - h + N_RING) % N_RING, r_recv.at[h - 1])
            send_descs.append(ring_send(
                (p - h + N_RING) % N_RING, r_send.at[h], r_recv.at[h], right))
            if h < H_L:
                if h > 0:
                    ring_recv_wait((p + h) % N_RING, l_recv.at[h - 1])
                send_descs.append(ring_send(
                    (p + h) % N_RING, l_send.at[h], l_recv.at[h], left))

        ring_recv_wait((p - H_R + N_RING) % N_RING, r_recv.at[H_R - 1])
        ring_recv_wait((p + H_L) % N_RING, l_recv.at[H_L - 1])
        for d in send_descs:
            d.wait_send()

    return pl.pallas_call(
        body,
        out_shape=jax.ShapeDtypeStruct((M_HALF, D), jnp.float32),
        in_specs=[
            pl.BlockSpec(memory_space=pl.ANY),
            pl.BlockSpec(memory_space=pltpu.MemorySpace.VMEM),
            pl.BlockSpec(memory_space=pltpu.MemorySpace.SMEM),
        ],
        out_specs=pl.BlockSpec(memory_space=pl.ANY),
        scratch_shapes=[
            pltpu.VMEM((SL, D), jnp.float32),
            pltpu.VMEM((SL, D), jnp.float32),
            pltpu.VMEM((SL, D), jnp.float32),
            pltpu.SemaphoreType.DMA,
            pltpu.SemaphoreType.DMA,
            pltpu.SemaphoreType.DMA,
            pltpu.SemaphoreType.DMA,
            pltpu.SemaphoreType.DMA((H_R,)),
            pltpu.SemaphoreType.DMA((H_R,)),
            pltpu.SemaphoreType.DMA((H_L,)),
            pltpu.SemaphoreType.DMA((H_L,)),
        ],
        compiler_params=pltpu.CompilerParams(collective_id=0),
    )(partial, gamma2d, scal)


# device time: 455496 ns/iter; 1.9109x vs baseline; 1.0518x over previous
import numpy as np
import jax
import jax.numpy as jnp
from jax import lax
from jax.experimental import pallas as pl
from jax.experimental.pallas import tpu as pltpu

M_HALF = 4096
D = 4096
N_RING = 16
SL = M_HALF // N_RING
SL2 = SL // 2
H = 8
EPS = 1e-6

_RING = [
    (0, 0), (0, 1), (0, 2), (0, 3),
    (1, 3), (1, 2), (1, 1),
    (2, 1), (2, 2), (2, 3),
    (3, 3), (3, 2), (3, 1), (3, 0),
    (2, 0), (1, 0),
]
_POS = np.zeros(16, dtype=np.int32)
for _i, (_y, _z) in enumerate(_RING):
    _POS[_y * 4 + _z] = _i
_RING_Y = np.array([y for y, _ in _RING], dtype=np.int32)
_RING_Z = np.array([z for _, z in _RING], dtype=np.int32)


def kernel(partial, gamma):
    gamma2d = gamma.reshape(1, D)

    my_y = lax.axis_index("y")
    my_z = lax.axis_index("z")
    p = jnp.take(jnp.asarray(_POS), my_y * 4 + my_z)
    rp = (p + 1) % N_RING
    lp = (p + N_RING - 1) % N_RING
    scal = jnp.stack([
        p,
        jnp.take(jnp.asarray(_RING_Y), rp), jnp.take(jnp.asarray(_RING_Z), rp),
        jnp.take(jnp.asarray(_RING_Y), lp), jnp.take(jnp.asarray(_RING_Z), lp),
    ]).astype(jnp.int32)

    def body(partial_ref, gamma_ref, scal_ref, out_ref,
             vmem_my, vmem_recv, vmem_out,
             pa_send, pa_recv, sem_in, sem_out,
             r_send, r_recv, l_send, l_recv):
        my_x = lax.axis_index("x")
        my_y = lax.axis_index("y")
        my_z = lax.axis_index("z")
        p = scal_ref[0]
        right = (my_x, scal_ref[1], scal_ref[2])
        left = (my_x, scal_ref[3], scal_ref[4])
        partner = (1 - my_x, my_y, my_z)

        barrier_sem = pltpu.get_barrier_semaphore()
        for nbr in (partner, right, left):
            pl.semaphore_signal(barrier_sem, inc=1, device_id=nbr,
                                device_id_type=pl.DeviceIdType.MESH)
        pl.semaphore_wait(barrier_sem, 3)

        rdma_a = pltpu.make_async_remote_copy(
            src_ref=partial_ref.at[0, pl.ds((1 - my_x) * M_HALF + p * SL, SL), :],
            dst_ref=vmem_recv,
            send_sem=pa_send,
            recv_sem=pa_recv,
            device_id=partner,
            device_id_type=pl.DeviceIdType.MESH,
        )
        rdma_a.start()
        cp_in = pltpu.make_async_copy(
            partial_ref.at[0, pl.ds(my_x * M_HALF + p * SL, SL), :],
            vmem_my, sem_in)
        cp_in.start()
        cp_in.wait()
        rdma_a.wait()

        s = vmem_my[:, :] + vmem_recv[:, :]
        r = lax.rsqrt(jnp.mean(s * s, axis=-1, keepdims=True) + EPS)
        vmem_out[:, :] = s * r * gamma_ref[:, :]
        cp_out = pltpu.make_async_copy(
            vmem_out, out_ref.at[pl.ds(p * SL, SL), :], sem_out)
        cp_out.start()

        def ring_send(src, dst_rows, sem_s, sem_r, dev):
            d = pltpu.make_async_remote_copy(
                src_ref=src,
                dst_ref=out_ref.at[dst_rows, :],
                send_sem=sem_s, recv_sem=sem_r,
                device_id=dev, device_id_type=pl.DeviceIdType.MESH,
            )
            d.start()
            return d

        def ring_recv_wait(rows, sem_r):
            d = pltpu.make_async_remote_copy(
                src_ref=out_ref.at[rows, :],
                dst_ref=out_ref.at[rows, :],
                send_sem=pa_send, recv_sem=sem_r,
                device_id=right, device_id_type=pl.DeviceIdType.MESH,
            )
            d.wait_recv()

        def rows_full(origin):
            return pl.ds(origin * SL, SL)

        def rows_half(origin, which):
            return pl.ds(origin * SL + which * SL2, SL2)

        send_descs = []
        for h in range(H):
            o_r = (p - h + N_RING) % N_RING
            o_l = (p + h) % N_RING
            if h == 0:
                send_descs.append(ring_send(
                    vmem_out, rows_full(p), r_send.at[0], r_recv.at[0], right))
                send_descs.append(ring_send(
                    vmem_out, rows_full(p), l_send.at[0], l_recv.at[0], left))
            elif h < H - 1:
                ring_recv_wait(rows_full(o_r), r_recv.at[h - 1])
                send_descs.append(ring_send(
                    out_ref.at[rows_full(o_r), :], rows_full(o_r),
                    r_send.at[h], r_recv.at[h], right))
                ring_recv_wait(rows_full(o_l), l_recv.at[h - 1])
                send_descs.append(ring_send(
                    out_ref.at[rows_full(o_l), :], rows_full(o_l),
                    l_send.at[h], l_recv.at[h], left))
            else:
                ring_recv_wait(rows_full(o_r), r_recv.at[h - 1])
                send_descs.append(ring_send(
                    out_ref.at[rows_half(o_r, 0), :], rows_half(o_r, 0),
                    r_send.at[h], r_recv.at[h], right))
                ring_recv_wait(rows_full(o_l), l_recv.at[h - 1])
                send_descs.append(ring_send(
                    out_ref.at[rows_half(o_l, 1), :], rows_half(o_l, 1),
                    l_send.at[h], l_recv.at[h], left))

        o_far = (p + H) % N_RING
        ring_recv_wait(rows_half(o_far, 0), r_recv.at[H - 1])
        ring_recv_wait(rows_half(o_far, 1), l_recv.at[H - 1])
        cp_out.wait()
        for d in send_descs:
            d.wait_send()

    return pl.pallas_call(
        body,
        out_shape=jax.ShapeDtypeStruct((M_HALF, D), jnp.float32),
        in_specs=[
            pl.BlockSpec(memory_space=pl.ANY),
            pl.BlockSpec(memory_space=pltpu.MemorySpace.VMEM),
            pl.BlockSpec(memory_space=pltpu.MemorySpace.SMEM),
        ],
        out_specs=pl.BlockSpec(memory_space=pl.ANY),
        scratch_shapes=[
            pltpu.VMEM((SL, D), jnp.float32),
            pltpu.VMEM((SL, D), jnp.float32),
            pltpu.VMEM((SL, D), jnp.float32),
            pltpu.SemaphoreType.DMA,
            pltpu.SemaphoreType.DMA,
            pltpu.SemaphoreType.DMA,
            pltpu.SemaphoreType.DMA,
            pltpu.SemaphoreType.DMA((H,)),
            pltpu.SemaphoreType.DMA((H,)),
            pltpu.SemaphoreType.DMA((H,)),
            pltpu.SemaphoreType.DMA((H,)),
        ],
        compiler_params=pltpu.CompilerParams(collective_id=0),
    )(partial, gamma2d, scal)


# device time: 384703 ns/iter; 2.2625x vs baseline; 1.1840x over previous
import numpy as np
import jax
import jax.numpy as jnp
from jax import lax
from jax.experimental import pallas as pl
from jax.experimental.pallas import tpu as pltpu

M_HALF = 4096
D = 4096
N_RING = 16
SL = M_HALF // N_RING
H = 6
N_EXTRA = 3
EPS = 1e-6

_RING = [
    (0, 0), (0, 1), (0, 2), (0, 3),
    (1, 3), (1, 2), (1, 1),
    (2, 1), (2, 2), (2, 3),
    (3, 3), (3, 2), (3, 1), (3, 0),
    (2, 0), (1, 0),
]
_POS = np.zeros(16, dtype=np.int32)
for _i, (_y, _z) in enumerate(_RING):
    _POS[_y * 4 + _z] = _i
_RING_Y = np.array([y for y, _ in _RING], dtype=np.int32)
_RING_Z = np.array([z for _, z in _RING], dtype=np.int32)


def kernel(partial, gamma):
    gamma2d = gamma.reshape(1, D)

    my_y = lax.axis_index("y")
    my_z = lax.axis_index("z")
    p = jnp.take(jnp.asarray(_POS), my_y * 4 + my_z)
    rp = (p + 1) % N_RING
    lp = (p + N_RING - 1) % N_RING
    scal = jnp.stack([
        p,
        jnp.take(jnp.asarray(_RING_Y), rp), jnp.take(jnp.asarray(_RING_Z), rp),
        jnp.take(jnp.asarray(_RING_Y), lp), jnp.take(jnp.asarray(_RING_Z), lp),
    ]).astype(jnp.int32)

    def body(partial_ref, gamma_ref, scal_ref, out_ref,
             vmem_my, vmem_recv, vmem_out, vmem_xout, extra_recv,
             pa_send, pa_recv, ex_send, ex_recv, sem_in, sem_out, sem_xout,
             r_send, r_recv, l_send, l_recv):
        my_x = lax.axis_index("x")
        my_y = lax.axis_index("y")
        my_z = lax.axis_index("z")
        p = scal_ref[0]
        right = (my_x, scal_ref[1], scal_ref[2])
        left = (my_x, scal_ref[3], scal_ref[4])
        partner = (1 - my_x, my_y, my_z)

        barrier_sem = pltpu.get_barrier_semaphore()
        for nbr in (partner, right, left):
            pl.semaphore_signal(barrier_sem, inc=1, device_id=nbr,
                                device_id_type=pl.DeviceIdType.MESH)
        pl.semaphore_wait(barrier_sem, 3)

        def extra_origin(k):
            return jnp.where(
                k == 0, (p + N_RING - 7) % N_RING,
                jnp.where(k == 1, (p + 7) % N_RING, (p + 8) % N_RING))

        rdma_a = pltpu.make_async_remote_copy(
            src_ref=partial_ref.at[0, pl.ds((1 - my_x) * M_HALF + p * SL, SL), :],
            dst_ref=vmem_recv,
            send_sem=pa_send,
            recv_sem=pa_recv,
            device_id=partner,
            device_id_type=pl.DeviceIdType.MESH,
        )
        rdma_a.start()
        extra_descs = []
        for k in range(N_EXTRA):
            o = extra_origin(k)
            d = pltpu.make_async_remote_copy(
                src_ref=partial_ref.at[
                    0, pl.ds((1 - my_x) * M_HALF + o * SL, SL), :],
                dst_ref=extra_recv.at[pl.ds(k * SL, SL), :],
                send_sem=ex_send.at[k],
                recv_sem=ex_recv.at[k],
                device_id=partner,
                device_id_type=pl.DeviceIdType.MESH,
            )
            d.start()
            extra_descs.append(d)

        cp_in = pltpu.make_async_copy(
            partial_ref.at[0, pl.ds(my_x * M_HALF + p * SL, SL), :],
            vmem_my, sem_in)
        cp_in.start()
        cp_in.wait()
        rdma_a.wait()

        def rmsnorm(a, b):
            s = a + b
            r = lax.rsqrt(jnp.mean(s * s, axis=-1, keepdims=True) + EPS)
            return s * r * gamma_ref[:, :]

        vmem_out[:, :] = rmsnorm(vmem_my[:, :], vmem_recv[:, :])
        cp_out = pltpu.make_async_copy(
            vmem_out, out_ref.at[pl.ds(p * SL, SL), :], sem_out)
        cp_out.start()

        def ring_send(src, dst_rows, sem_s, sem_r, dev):
            d = pltpu.make_async_remote_copy(
                src_ref=src,
                dst_ref=out_ref.at[dst_rows, :],
                send_sem=sem_s, recv_sem=sem_r,
                device_id=dev, device_id_type=pl.DeviceIdType.MESH,
            )
            d.start()
            return d

        def ring_recv_wait(rows, sem_r):
            d = pltpu.make_async_remote_copy(
                src_ref=out_ref.at[rows, :],
                dst_ref=out_ref.at[rows, :],
                send_sem=pa_send, recv_sem=sem_r,
                device_id=right, device_id_type=pl.DeviceIdType.MESH,
            )
            d.wait_recv()

        def rows_full(origin):
            return pl.ds(origin * SL, SL)

        send_descs = []
        for h in range(H):
            o_r = (p - h + N_RING) % N_RING
            o_l = (p + h) % N_RING
            if h == 0:
                send_descs.append(ring_send(
                    vmem_out, rows_full(p), r_send.at[0], r_recv.at[0], right))
                send_descs.append(ring_send(
                    vmem_out, rows_full(p), l_send.at[0], l_recv.at[0], left))
            else:
                ring_recv_wait(rows_full(o_r), r_recv.at[h - 1])
                send_descs.append(ring_send(
                    out_ref.at[rows_full(o_r), :], rows_full(o_r),
                    r_send.at[h], r_recv.at[h], right))
                ring_recv_wait(rows_full(o_l), l_recv.at[h - 1])
                send_descs.append(ring_send(
                    out_ref.at[rows_full(o_l), :], rows_full(o_l),
                    l_send.at[h], l_recv.at[h], left))

        prev_store = None
        for k in range(N_EXTRA):
            o = extra_origin(k)
            cp_k = pltpu.make_async_copy(
                partial_ref.at[0, pl.ds(my_x * M_HALF + o * SL, SL), :],
                vmem_my, sem_in)
            cp_k.start()
            extra_descs[k].wait()
            cp_k.wait()
            if prev_store is not None:
                prev_store.wait()
            vmem_xout[:, :] = rmsnorm(
                vmem_my[:, :], extra_recv[pl.ds(k * SL, SL), :])
            prev_store = pltpu.make_async_copy(
                vmem_xout, out_ref.at[pl.ds(o * SL, SL), :], sem_xout)
            prev_store.start()
        prev_store.wait()

        ring_recv_wait(rows_full((p - H + N_RING) % N_RING), r_recv.at[H - 1])
        ring_recv_wait(rows_full((p + H) % N_RING), l_recv.at[H - 1])
        cp_out.wait()
        for d in send_descs:
            d.wait_send()

    return pl.pallas_call(
        body,
        out_shape=jax.ShapeDtypeStruct((M_HALF, D), jnp.float32),
        in_specs=[
            pl.BlockSpec(memory_space=pl.ANY),
            pl.BlockSpec(memory_space=pltpu.MemorySpace.VMEM),
            pl.BlockSpec(memory_space=pltpu.MemorySpace.SMEM),
        ],
        out_specs=pl.BlockSpec(memory_space=pl.ANY),
        scratch_shapes=[
            pltpu.VMEM((SL, D), jnp.float32),
            pltpu.VMEM((SL, D), jnp.float32),
            pltpu.VMEM((SL, D), jnp.float32),
            pltpu.VMEM((SL, D), jnp.float32),
            pltpu.VMEM((N_EXTRA * SL, D), jnp.float32),
            pltpu.SemaphoreType.DMA,
            pltpu.SemaphoreType.DMA,
            pltpu.SemaphoreType.DMA((N_EXTRA,)),
            pltpu.SemaphoreType.DMA((N_EXTRA,)),
            pltpu.SemaphoreType.DMA,
            pltpu.SemaphoreType.DMA,
            pltpu.SemaphoreType.DMA,
            pltpu.SemaphoreType.DMA((H,)),
            pltpu.SemaphoreType.DMA((H,)),
            pltpu.SemaphoreType.DMA((H,)),
            pltpu.SemaphoreType.DMA((H,)),
        ],
        compiler_params=pltpu.CompilerParams(collective_id=0),
    )(partial, gamma2d, scal)


# device time: 363481 ns/iter; 2.3946x vs baseline; 1.0584x over previous
import numpy as np
import jax
import jax.numpy as jnp
from jax import lax
from jax.experimental import pallas as pl
from jax.experimental.pallas import tpu as pltpu

M_HALF = 4096
D = 4096
N_RING = 16
SL = M_HALF // N_RING
SL2 = SL // 2
H = 6
N_EXTRA = 5
EPS = 1e-6

_EXTRAS = [(-7, 0, SL), (7, 0, SL), (8, 0, SL), (-6, SL2, SL2), (6, 0, SL2)]
_EXTRA_BUF_OFF = [0, SL, 2 * SL, 3 * SL, 3 * SL + SL2]
_EXTRA_BUF_ROWS = 4 * SL

_RING = [
    (0, 0), (0, 1), (0, 2), (0, 3),
    (1, 3), (1, 2), (1, 1),
    (2, 1), (2, 2), (2, 3),
    (3, 3), (3, 2), (3, 1), (3, 0),
    (2, 0), (1, 0),
]
_POS = np.zeros(16, dtype=np.int32)
for _i, (_y, _z) in enumerate(_RING):
    _POS[_y * 4 + _z] = _i
_RING_Y = np.array([y for y, _ in _RING], dtype=np.int32)
_RING_Z = np.array([z for _, z in _RING], dtype=np.int32)


def kernel(partial, gamma):
    gamma2d = gamma.reshape(1, D)

    my_y = lax.axis_index("y")
    my_z = lax.axis_index("z")
    p = jnp.take(jnp.asarray(_POS), my_y * 4 + my_z)
    rp = (p + 1) % N_RING
    lp = (p + N_RING - 1) % N_RING
    scal = jnp.stack([
        p,
        jnp.take(jnp.asarray(_RING_Y), rp), jnp.take(jnp.asarray(_RING_Z), rp),
        jnp.take(jnp.asarray(_RING_Y), lp), jnp.take(jnp.asarray(_RING_Z), lp),
    ]).astype(jnp.int32)

    def body(partial_ref, gamma_ref, scal_ref, out_ref,
             vmem_my, vmem_recv, vmem_out, vmem_xout, extra_recv,
             pa_send, pa_recv, ex_send, ex_recv, sem_in, sem_out, sem_xout,
             r_send, r_recv, l_send, l_recv):
        my_x = lax.axis_index("x")
        my_y = lax.axis_index("y")
        my_z = lax.axis_index("z")
        p = scal_ref[0]
        right = (my_x, scal_ref[1], scal_ref[2])
        left = (my_x, scal_ref[3], scal_ref[4])
        partner = (1 - my_x, my_y, my_z)

        barrier_sem = pltpu.get_barrier_semaphore()
        for nbr in (partner, right, left):
            pl.semaphore_signal(barrier_sem, inc=1, device_id=nbr,
                                device_id_type=pl.DeviceIdType.MESH)
        pl.semaphore_wait(barrier_sem, 3)

        def origin_of(off):
            return (p + off + N_RING) % N_RING

        rdma_a = pltpu.make_async_remote_copy(
            src_ref=partial_ref.at[0, pl.ds((1 - my_x) * M_HALF + p * SL, SL), :],
            dst_ref=vmem_recv,
            send_sem=pa_send,
            recv_sem=pa_recv,
            device_id=partner,
            device_id_type=pl.DeviceIdType.MESH,
        )
        rdma_a.start()
        extra_descs = []
        for k, (off, roff, nrows) in enumerate(_EXTRAS):
            o = origin_of(off)
            d = pltpu.make_async_remote_copy(
                src_ref=partial_ref.at[
                    0, pl.ds((1 - my_x) * M_HALF + o * SL + roff, nrows), :],
                dst_ref=extra_recv.at[pl.ds(_EXTRA_BUF_OFF[k], nrows), :],
                send_sem=ex_send.at[k],
                recv_sem=ex_recv.at[k],
                device_id=partner,
                device_id_type=pl.DeviceIdType.MESH,
            )
            d.start()
            extra_descs.append(d)

        cp_in = pltpu.make_async_copy(
            partial_ref.at[0, pl.ds(my_x * M_HALF + p * SL, SL), :],
            vmem_my, sem_in)
        cp_in.start()
        cp_in.wait()
        rdma_a.wait()

        def rmsnorm(a, b):
            s = a + b
            r = lax.rsqrt(jnp.mean(s * s, axis=-1, keepdims=True) + EPS)
            return s * r * gamma_ref[:, :]

        vmem_out[:, :] = rmsnorm(vmem_my[:, :], vmem_recv[:, :])
        cp_out = pltpu.make_async_copy(
            vmem_out, out_ref.at[pl.ds(p * SL, SL), :], sem_out)
        cp_out.start()

        def ring_send(src, dst_rows, sem_s, sem_r, dev):
            d = pltpu.make_async_remote_copy(
                src_ref=src,
                dst_ref=out_ref.at[dst_rows, :],
                send_sem=sem_s, recv_sem=sem_r,
                device_id=dev, device_id_type=pl.DeviceIdType.MESH,
            )
            d.start()
            return d

        def ring_recv_wait(rows, sem_r):
            d = pltpu.make_async_remote_copy(
                src_ref=out_ref.at[rows, :],
                dst_ref=out_ref.at[rows, :],
                send_sem=pa_send, recv_sem=sem_r,
                device_id=right, device_id_type=pl.DeviceIdType.MESH,
            )
            d.wait_recv()

        def rows_full(origin):
            return pl.ds(origin * SL, SL)

        def do_extra(k, prev_store):
            off, roff, nrows = _EXTRAS[k]
            o = origin_of(off)
            cp_k = pltpu.make_async_copy(
                partial_ref.at[0, pl.ds(my_x * M_HALF + o * SL + roff, nrows), :],
                vmem_my.at[pl.ds(0, nrows), :], sem_in)
            cp_k.start()
            extra_descs[k].wait()
            cp_k.wait()
            if prev_store is not None:
                prev_store.wait()
            vmem_xout[pl.ds(0, nrows), :] = rmsnorm(
                vmem_my[pl.ds(0, nrows), :],
                extra_recv[pl.ds(_EXTRA_BUF_OFF[k], nrows), :])
            st = pltpu.make_async_copy(
                vmem_xout.at[pl.ds(0, nrows), :],
                out_ref.at[pl.ds(o * SL + roff, nrows), :], sem_xout)
            st.start()
            return st

        send_descs = []
        prev_store = None
        for h in range(H):
            o_r = (p - h + N_RING) % N_RING
            o_l = (p + h) % N_RING
            if h == 0:
                send_descs.append(ring_send(
                    vmem_out, rows_full(p), r_send.at[0], r_recv.at[0], right))
                send_descs.append(ring_send(
                    vmem_out, rows_full(p), l_send.at[0], l_recv.at[0], left))
            elif h < H - 1:
                ring_recv_wait(rows_full(o_r), r_recv.at[h - 1])
                send_descs.append(ring_send(
                    out_ref.at[rows_full(o_r), :], rows_full(o_r),
                    r_send.at[h], r_recv.at[h], right))
                ring_recv_wait(rows_full(o_l), l_recv.at[h - 1])
                send_descs.append(ring_send(
                    out_ref.at[rows_full(o_l), :], rows_full(o_l),
                    l_send.at[h], l_recv.at[h], left))
            else:
                ring_recv_wait(rows_full(o_r), r_recv.at[h - 1])
                send_descs.append(ring_send(
                    out_ref.at[pl.ds(o_r * SL, SL2), :],
                    pl.ds(o_r * SL, SL2),
                    r_send.at[h], r_recv.at[h], right))
                ring_recv_wait(rows_full(o_l), l_recv.at[h - 1])
                send_descs.append(ring_send(
                    out_ref.at[pl.ds(o_l * SL + SL2, SL2), :],
                    pl.ds(o_l * SL + SL2, SL2),
                    l_send.at[h], l_recv.at[h], left))
            if 1 <= h:
                prev_store = do_extra(h - 1, prev_store)

        o_r6 = (p - H + N_RING) % N_RING
        o_l6 = (p + H) % N_RING
        ring_recv_wait(pl.ds(o_r6 * SL, SL2), r_recv.at[H - 1])
        ring_recv_wait(pl.ds(o_l6 * SL + SL2, SL2), l_recv.at[H - 1])
        prev_store.wait()
        cp_out.wait()
        for d in send_descs:
            d.wait_send()

    return pl.pallas_call(
        body,
        out_shape=jax.ShapeDtypeStruct((M_HALF, D), jnp.float32),
        in_specs=[
            pl.BlockSpec(memory_space=pl.ANY),
            pl.BlockSpec(memory_space=pltpu.MemorySpace.VMEM),
            pl.BlockSpec(memory_space=pltpu.MemorySpace.SMEM),
        ],
        out_specs=pl.BlockSpec(memory_space=pl.ANY),
        scratch_shapes=[
            pltpu.VMEM((SL, D), jnp.float32),
            pltpu.VMEM((SL, D), jnp.float32),
            pltpu.VMEM((SL, D), jnp.float32),
            pltpu.VMEM((SL, D), jnp.float32),
            pltpu.VMEM((_EXTRA_BUF_ROWS, D), jnp.float32),
            pltpu.SemaphoreType.DMA,
            pltpu.SemaphoreType.DMA,
            pltpu.SemaphoreType.DMA((N_EXTRA,)),
            pltpu.SemaphoreType.DMA((N_EXTRA,)),
            pltpu.SemaphoreType.DMA,
            pltpu.SemaphoreType.DMA,
            pltpu.SemaphoreType.DMA,
            pltpu.SemaphoreType.DMA((H,)),
            pltpu.SemaphoreType.DMA((H,)),
            pltpu.SemaphoreType.DMA((H,)),
            pltpu.SemaphoreType.DMA((H,)),
        ],
        compiler_params=pltpu.CompilerParams(
            collective_id=0, vmem_limit_bytes=64 * 1024 * 1024),
    )(partial, gamma2d, scal)


# device time: 340508 ns/iter; 2.5562x vs baseline; 1.0675x over previous
import numpy as np
import jax
import jax.numpy as jnp
from jax import lax
from jax.experimental import pallas as pl
from jax.experimental.pallas import tpu as pltpu

M_HALF = 4096
D = 4096
N_RING = 16
SL = M_HALF // N_RING
SL2 = SL // 2
H = 6
N_EXTRA = 5
EPS = 1e-6

_EXTRAS = [(-7, 0, SL), (7, 0, SL), (8, 0, SL), (-6, SL2, SL2), (6, 0, SL2)]
_EXTRA_BUF_OFF = [0, SL, 2 * SL, 3 * SL, 3 * SL + SL2]
_EXTRA_BUF_ROWS = 4 * SL

_RING = [
    (0, 0), (0, 1), (0, 2), (0, 3),
    (1, 3), (1, 2), (1, 1),
    (2, 1), (2, 2), (2, 3),
    (3, 3), (3, 2), (3, 1), (3, 0),
    (2, 0), (1, 0),
]
_POS = np.zeros(16, dtype=np.int32)
for _i, (_y, _z) in enumerate(_RING):
    _POS[_y * 4 + _z] = _i
_RING_Y = np.array([y for y, _ in _RING], dtype=np.int32)
_RING_Z = np.array([z for _, z in _RING], dtype=np.int32)


def kernel(partial, gamma):
    gamma2d = gamma.reshape(1, D)

    my_y = lax.axis_index("y")
    my_z = lax.axis_index("z")
    p = jnp.take(jnp.asarray(_POS), my_y * 4 + my_z)
    rp = (p + 1) % N_RING
    lp = (p + N_RING - 1) % N_RING
    scal = jnp.stack([
        p,
        jnp.take(jnp.asarray(_RING_Y), rp), jnp.take(jnp.asarray(_RING_Z), rp),
        jnp.take(jnp.asarray(_RING_Y), lp), jnp.take(jnp.asarray(_RING_Z), lp),
    ]).astype(jnp.int32)

    def body(partial_ref, gamma_ref, scal_ref, out_ref,
             vmem_my, vmem_recv, vmem_out, vmem_xout, extra_recv,
             pa_send, pa_recv, ex_send, ex_recv, sem_in, sem_out, sem_xout,
             r_send, r_recv, l_send, l_recv):
        my_x = lax.axis_index("x")
        my_y = lax.axis_index("y")
        my_z = lax.axis_index("z")
        p = scal_ref[0]
        right = (my_x, scal_ref[1], scal_ref[2])
        left = (my_x, scal_ref[3], scal_ref[4])
        partner = (1 - my_x, my_y, my_z)

        barrier_sem = pltpu.get_barrier_semaphore()
        for nbr in (partner, right, left):
            pl.semaphore_signal(barrier_sem, inc=1, device_id=nbr,
                                device_id_type=pl.DeviceIdType.MESH)
        pl.semaphore_wait(barrier_sem, 3)

        def origin_of(off):
            return (p + off + N_RING) % N_RING

        rdma_a = []
        for j in range(2):
            d = pltpu.make_async_remote_copy(
                src_ref=partial_ref.at[
                    0, pl.ds((1 - my_x) * M_HALF + p * SL + j * SL2, SL2), :],
                dst_ref=vmem_recv.at[pl.ds(j * SL2, SL2), :],
                send_sem=pa_send.at[j],
                recv_sem=pa_recv.at[j],
                device_id=partner,
                device_id_type=pl.DeviceIdType.MESH,
            )
            d.start()
            rdma_a.append(d)
        extra_descs = []
        for k, (off, roff, nrows) in enumerate(_EXTRAS):
            o = origin_of(off)
            d = pltpu.make_async_remote_copy(
                src_ref=partial_ref.at[
                    0, pl.ds((1 - my_x) * M_HALF + o * SL + roff, nrows), :],
                dst_ref=extra_recv.at[pl.ds(_EXTRA_BUF_OFF[k], nrows), :],
                send_sem=ex_send.at[k],
                recv_sem=ex_recv.at[k],
                device_id=partner,
                device_id_type=pl.DeviceIdType.MESH,
            )
            d.start()
            extra_descs.append(d)

        cp_in = pltpu.make_async_copy(
            partial_ref.at[0, pl.ds(my_x * M_HALF + p * SL, SL), :],
            vmem_my, sem_in)
        cp_in.start()

        def rmsnorm(a, b):
            s = a + b
            r = lax.rsqrt(jnp.mean(s * s, axis=-1, keepdims=True) + EPS)
            return s * r * gamma_ref[:, :]

        def ring_send(src, dst_rows, sem_s, sem_r, dev):
            d = pltpu.make_async_remote_copy(
                src_ref=src,
                dst_ref=out_ref.at[dst_rows, :],
                send_sem=sem_s, recv_sem=sem_r,
                device_id=dev, device_id_type=pl.DeviceIdType.MESH,
            )
            d.start()
            return d

        def ring_recv_wait(rows, sem_r):
            d = pltpu.make_async_remote_copy(
                src_ref=out_ref.at[rows, :],
                dst_ref=out_ref.at[rows, :],
                send_sem=pa_send.at[0], recv_sem=sem_r,
                device_id=right, device_id_type=pl.DeviceIdType.MESH,
            )
            d.wait_recv()

        def rows_full(origin):
            return pl.ds(origin * SL, SL)

        def do_extra(k, prev_store):
            off, roff, nrows = _EXTRAS[k]
            o = origin_of(off)
            cp_k = pltpu.make_async_copy(
                partial_ref.at[0, pl.ds(my_x * M_HALF + o * SL + roff, nrows), :],
                vmem_my.at[pl.ds(0, nrows), :], sem_in)
            cp_k.start()
            extra_descs[k].wait()
            cp_k.wait()
            if prev_store is not None:
                prev_store.wait()
            vmem_xout[pl.ds(0, nrows), :] = rmsnorm(
                vmem_my[pl.ds(0, nrows), :],
                extra_recv[pl.ds(_EXTRA_BUF_OFF[k], nrows), :])
            st = pltpu.make_async_copy(
                vmem_xout.at[pl.ds(0, nrows), :],
                out_ref.at[pl.ds(o * SL + roff, nrows), :], sem_xout)
            st.start()
            return st

        send_descs = []

        cp_in.wait()
        for j in range(2):
            half = pl.ds(j * SL2, SL2)
            rdma_a[j].wait()
            vmem_out[half, :] = rmsnorm(vmem_my[half, :], vmem_recv[half, :])
            dst = pl.ds(p * SL + j * SL2, SL2)
            send_descs.append(ring_send(
                vmem_out.at[half, :], dst, r_send.at[j], r_recv.at[j], right))
            send_descs.append(ring_send(
                vmem_out.at[half, :], dst, l_send.at[j], l_recv.at[j], left))
        cp_out = pltpu.make_async_copy(
            vmem_out, out_ref.at[pl.ds(p * SL, SL), :], sem_out)
        cp_out.start()

        prev_store = None
        for h in range(1, H):
            o_r = (p - h + N_RING) % N_RING
            o_l = (p + h) % N_RING
            if h == 1:
                ring_recv_wait(pl.ds(o_r * SL, SL2), r_recv.at[0])
                ring_recv_wait(pl.ds(o_r * SL + SL2, SL2), r_recv.at[1])
            else:
                ring_recv_wait(rows_full(o_r), r_recv.at[h])
            if h < H - 1:
                send_descs.append(ring_send(
                    out_ref.at[rows_full(o_r), :], rows_full(o_r),
                    r_send.at[h + 1], r_recv.at[h + 1], right))
            else:
                send_descs.append(ring_send(
                    out_ref.at[pl.ds(o_r * SL, SL2), :],
                    pl.ds(o_r * SL, SL2),
                    r_send.at[h + 1], r_recv.at[h + 1], right))
            if h == 1:
                ring_recv_wait(pl.ds(o_l * SL, SL2), l_recv.at[0])
                ring_recv_wait(pl.ds(o_l * SL + SL2, SL2), l_recv.at[1])
            else:
                ring_recv_wait(rows_full(o_l), l_recv.at[h])
            if h < H - 1:
                send_descs.append(ring_send(
                    out_ref.at[rows_full(o_l), :], rows_full(o_l),
                    l_send.at[h + 1], l_recv.at[h + 1], left))
            else:
                send_descs.append(ring_send(
                    out_ref.at[pl.ds(o_l * SL + SL2, SL2), :],
                    pl.ds(o_l * SL + SL2, SL2),
                    l_send.at[h + 1], l_recv.at[h + 1], left))
            prev_store = do_extra(h - 1, prev_store)

        o_r6 = (p - H + N_RING) % N_RING
        o_l6 = (p + H) % N_RING
        ring_recv_wait(pl.ds(o_r6 * SL, SL2), r_recv.at[H])
        ring_recv_wait(pl.ds(o_l6 * SL + SL2, SL2), l_recv.at[H])
        prev_store.wait()
        cp_out.wait()
        for d in send_descs:
            d.wait_send()

    return pl.pallas_call(
        body,
        out_shape=jax.ShapeDtypeStruct((M_HALF, D), jnp.float32),
        in_specs=[
            pl.BlockSpec(memory_space=pl.ANY),
            pl.BlockSpec(memory_space=pltpu.MemorySpace.VMEM),
            pl.BlockSpec(memory_space=pltpu.MemorySpace.SMEM),
        ],
        out_specs=pl.BlockSpec(memory_space=pl.ANY),
        scratch_shapes=[
            pltpu.VMEM((SL, D), jnp.float32),
            pltpu.VMEM((SL, D), jnp.float32),
            pltpu.VMEM((SL, D), jnp.float32),
            pltpu.VMEM((SL, D), jnp.float32),
            pltpu.VMEM((_EXTRA_BUF_ROWS, D), jnp.float32),
            pltpu.SemaphoreType.DMA((2,)),
            pltpu.SemaphoreType.DMA((2,)),
            pltpu.SemaphoreType.DMA((N_EXTRA,)),
            pltpu.SemaphoreType.DMA((N_EXTRA,)),
            pltpu.SemaphoreType.DMA,
            pltpu.SemaphoreType.DMA,
            pltpu.SemaphoreType.DMA,
            pltpu.SemaphoreType.DMA((H + 1,)),
            pltpu.SemaphoreType.DMA((H + 1,)),
            pltpu.SemaphoreType.DMA((H + 1,)),
            pltpu.SemaphoreType.DMA((H + 1,)),
        ],
        compiler_params=pltpu.CompilerParams(
            collective_id=0, vmem_limit_bytes=64 * 1024 * 1024),
    )(partial, gamma2d, scal)
